# baseline (speedup 1.0000x reference)
import jax
import jax.numpy as jnp
from jax import lax
from jax.experimental import pallas as pl
from jax.experimental.pallas import tpu as pltpu
from jax.experimental.pallas import tpu_sc as plsc

BATCH = 16384
DIM = 64
NW = 32
B_PER_W = BATCH // NW
ROWS_PER_W = 2 * B_PER_W
CHUNK = 128
NCHUNK = ROWS_PER_W // CHUNK
E_PER_CHUNK = CHUNK // 2
NBLK = B_PER_W // 16


def _rsqrt_newton(s):
    i = plsc.bitcast(s, jnp.int32)
    y = plsc.bitcast(jnp.int32(0x5F3759DF) - (i >> 1), jnp.float32)
    half = s * 0.5
    for _ in range(3):
        y = y * (1.5 - half * y * y)
    return jnp.minimum(y, 1e12)


def _body(xr_hbm, xh_hbm, w_hbm, out_hbm, idx_v, half_v, buf0, buf1,
          sums_v, out_v, sem0, sem1):
    wid = lax.axis_index("s") * 2 + lax.axis_index("c")

    pltpu.sync_copy(xr_hbm.at[wid], idx_v)
    pltpu.sync_copy(xh_hbm.at[wid], half_v)

    bufs = (buf0, buf1)
    sems = (sem0, sem1)

    def gather(j):
        return pltpu.async_copy(
            w_hbm.at[idx_v.at[j]], bufs[j % 2], sems[j % 2]
        )

    lanes = lax.iota(jnp.int32, 16)
    last = lanes == 15

    def compute_chunk(j, buf):
        def e_body(i, _):
            p = j * CHUNK + 2 * i
            h0 = plsc.load_gather(half_v, [jnp.full((16,), 0, jnp.int32) + p])
            h1 = plsc.load_gather(
                half_v, [jnp.full((16,), 0, jnp.int32) + (p + 1)]
            )
            c0 = h0 * 64 + lanes
            c1 = h1 * 64 + lanes
            r0 = jnp.full((16,), 0, jnp.int32) + 2 * i
            r1 = r0 + 1
            p_acc = jnp.zeros((16,), jnp.float32)
            q_acc = jnp.zeros((16,), jnp.float32)
            r_acc = jnp.zeros((16,), jnp.float32)
            for k in range(4):
                a = plsc.load_gather(buf, [r0, c0 + k * 16])
                b = plsc.load_gather(buf, [r1, c1 + k * 16])
                p_acc = p_acc + a * b
                q_acc = q_acc + a * a
                r_acc = r_acc + b * b
            ei = jnp.full((16,), 0, jnp.int32) + (j * E_PER_CHUNK + i)
            plsc.store_scatter(sums_v, [ei], plsc.cumsum(p_acc), mask=last)
            plsc.store_scatter(
                sums_v, [ei + B_PER_W], plsc.cumsum(q_acc), mask=last)
            plsc.store_scatter(
                sums_v, [ei + 2 * B_PER_W], plsc.cumsum(r_acc), mask=last)
            return 0

        lax.fori_loop(0, E_PER_CHUNK, e_body, 0, unroll=2)

    copies = [gather(0)]
    for j in range(NCHUNK):
        if j + 1 < NCHUNK:
            copies.append(gather(j + 1))
        copies[j].wait()
        compute_chunk(j, bufs[j % 2])

    def blk_body(blk, _):
        sl = pl.ds(blk * 16, 16)
        s01 = sums_v[sl]
        s00 = sums_v[pl.ds(B_PER_W + blk * 16, 16)]
        s11 = sums_v[pl.ds(2 * B_PER_W + blk * 16, 16)]
        out_v[sl] = s01 * _rsqrt_newton(s00) * _rsqrt_newton(s11)
        return 0

    lax.fori_loop(0, NBLK, blk_body, 0)

    pltpu.sync_copy(out_v, out_hbm.at[pl.ds(wid * B_PER_W, B_PER_W)])


def kernel(x, W):
    xi = x.astype(jnp.int32)
    xr3 = (xi >> 1).reshape(NW, NCHUNK, CHUNK)
    xh3 = (xi & 1).reshape(NW, NCHUNK * CHUNK)
    W2 = W.reshape(500000, 128)
    mesh = plsc.VectorSubcoreMesh(core_axis_name="c", subcore_axis_name="s")
    out = pl.kernel(
        _body,
        mesh=mesh,
        compiler_params=pltpu.CompilerParams(
            needs_layout_passes=False, use_tc_tiling_on_sc=True
        ),
        out_type=jax.ShapeDtypeStruct((BATCH,), jnp.float32),
        scratch_types=[
            pltpu.VMEM((NCHUNK, CHUNK), jnp.int32),
            pltpu.VMEM((NCHUNK * CHUNK,), jnp.int32),
            pltpu.VMEM((CHUNK, CHUNK), jnp.float32),
            pltpu.VMEM((CHUNK, CHUNK), jnp.float32),
            pltpu.VMEM((3 * B_PER_W,), jnp.float32),
            pltpu.VMEM((B_PER_W,), jnp.float32),
            pltpu.SemaphoreType.DMA,
            pltpu.SemaphoreType.DMA,
        ],
    )(xr3, xh3, W2)
    return out[:, None]


# speedup vs baseline: 1.6095x; 1.6095x over previous
import jax
import jax.numpy as jnp
from jax import lax
from jax.experimental import pallas as pl
from jax.experimental.pallas import tpu as pltpu
from jax.experimental.pallas import tpu_sc as plsc

BATCH = 16384
DIM = 64
NW = 32
B_PER_W = BATCH // NW
ROWS_PER_W = 2 * B_PER_W
CHUNK = 128
NCHUNK = ROWS_PER_W // CHUNK
E_PER_CHUNK = CHUNK // 2
NBLK = B_PER_W // 16


def _rsqrt_newton(s):
    i = plsc.bitcast(s, jnp.int32)
    y = plsc.bitcast(jnp.int32(0x5F3759DF) - (i >> 1), jnp.float32)
    half = s * 0.5
    for _ in range(3):
        y = y * (1.5 - half * y * y)
    return jnp.minimum(y, 1e12)


def _body(x_hbm, w_hbm, out_hbm, idx_v, buf0, buf1, sums_v, out_v,
          sem0, sem1):
    wid = lax.axis_index("s") * 2 + lax.axis_index("c")

    pltpu.sync_copy(x_hbm.at[wid], idx_v)

    bufs = (buf0, buf1)
    sems = (sem0, sem1)
    zero16 = jnp.full((16,), 0, jnp.int32)

    def fire_chunk(j):
        buf, sem = bufs[j % 2], sems[j % 2]

        def row_body(slot, _):
            rv = plsc.load_gather(idx_v, [zero16 + (j * CHUNK + slot)])
            r = rv[0]
            pltpu.async_copy(
                w_hbm.at[r],
                buf.at[slot, pl.ds(0, DIM)],
                sem,
            )
            return 0

        lax.fori_loop(0, CHUNK, row_body, 0, unroll=4)

    def drain_chunk(j):
        buf, sem = bufs[j % 2], sems[j % 2]

        def wait_body(_, c):
            pltpu.make_async_copy(
                w_hbm.at[0], buf.at[0, pl.ds(0, DIM)], sem
            ).wait()
            return c

        lax.fori_loop(0, CHUNK, wait_body, 0)

    lanes = lax.iota(jnp.int32, 16)
    last = lanes == 15

    def compute_chunk(j, buf):
        def e_body(i, _):
            p_acc = jnp.zeros((16,), jnp.float32)
            q_acc = jnp.zeros((16,), jnp.float32)
            r_acc = jnp.zeros((16,), jnp.float32)
            for k in range(4):
                a = buf[2 * i, pl.ds(k * 16, 16)]
                b = buf[2 * i + 1, pl.ds(k * 16, 16)]
                p_acc = p_acc + a * b
                q_acc = q_acc + a * a
                r_acc = r_acc + b * b
            ei = zero16 + (j * E_PER_CHUNK + i)
            plsc.store_scatter(sums_v, [ei], plsc.cumsum(p_acc), mask=last)
            plsc.store_scatter(
                sums_v, [ei + B_PER_W], plsc.cumsum(q_acc), mask=last)
            plsc.store_scatter(
                sums_v, [ei + 2 * B_PER_W], plsc.cumsum(r_acc), mask=last)
            return 0

        lax.fori_loop(0, E_PER_CHUNK, e_body, 0, unroll=2)

    fire_chunk(0)
    for j in range(NCHUNK):
        if j + 1 < NCHUNK:
            fire_chunk(j + 1)
        drain_chunk(j)
        compute_chunk(j, bufs[j % 2])

    def blk_body(blk, _):
        sl = pl.ds(blk * 16, 16)
        s01 = sums_v[sl]
        s00 = sums_v[pl.ds(B_PER_W + blk * 16, 16)]
        s11 = sums_v[pl.ds(2 * B_PER_W + blk * 16, 16)]
        out_v[sl] = s01 * _rsqrt_newton(s00) * _rsqrt_newton(s11)
        return 0

    lax.fori_loop(0, NBLK, blk_body, 0)

    pltpu.sync_copy(out_v, out_hbm.at[pl.ds(wid * B_PER_W, B_PER_W)])


def kernel(x, W):
    x3 = x.astype(jnp.int32).reshape(NW, ROWS_PER_W)
    mesh = plsc.VectorSubcoreMesh(core_axis_name="c", subcore_axis_name="s")
    out = pl.kernel(
        _body,
        mesh=mesh,
        compiler_params=pltpu.CompilerParams(
            needs_layout_passes=False, use_tc_tiling_on_sc=True
        ),
        out_type=jax.ShapeDtypeStruct((BATCH,), jnp.float32),
        scratch_types=[
            pltpu.VMEM((ROWS_PER_W,), jnp.int32),
            pltpu.VMEM((CHUNK, CHUNK), jnp.float32),
            pltpu.VMEM((CHUNK, CHUNK), jnp.float32),
            pltpu.VMEM((3 * B_PER_W,), jnp.float32),
            pltpu.VMEM((B_PER_W,), jnp.float32),
            pltpu.SemaphoreType.DMA,
            pltpu.SemaphoreType.DMA,
        ],
    )(x3, W)
    return out[:, None]


# speedup vs baseline: 2.4642x; 1.5311x over previous
import jax
import jax.numpy as jnp
from jax import lax
from jax.experimental import pallas as pl
from jax.experimental.pallas import tpu as pltpu
from jax.experimental.pallas import tpu_sc as plsc

BATCH = 16384
DIM = 64
NUMS = 1000000
NW = 32
B_PER_W = BATCH // NW
L_PER_W = 2 * B_PER_W
TC_EDGE = NUMS // 128
EDGE0 = TC_EDGE * 128
CHUNK2 = 256
NCHUNK2 = L_PER_W // CHUNK2
NBLK = B_PER_W // 16


def _rsqrt_newton(s):
    i = plsc.bitcast(s, jnp.int32)
    y = plsc.bitcast(jnp.int32(0x5F3759DF) - (i >> 1), jnp.float32)
    half = s * 0.5
    for _ in range(3):
        y = y * (1.5 - half * y * y)
    return jnp.minimum(y, 1e12)


def _gather_body(sv_hbm, pv_hbm, wt_hbm, wedge_hbm, stage_hbm,
                 sv, pv, ev, tiles, tmp, sem0, sem1, osem):
    wid = lax.axis_index("s") * 2 + lax.axis_index("c")

    pltpu.sync_copy(sv_hbm.at[wid], sv)
    pltpu.sync_copy(pv_hbm.at[wid], pv)
    pltpu.sync_copy(wedge_hbm, ev)

    zero16 = jnp.full((16,), 0, jnp.int32)
    lanes = lax.iota(jnp.int32, 16)

    def sval(q):
        return plsc.load_gather(sv, [zero16 + q])[0]

    def pval(q):
        return plsc.load_gather(pv, [zero16 + q])[0]

    def fire(tcv):
        src = wt_hbm.at[:, pl.ds(pl.multiple_of(tcv * 128, 128), 128)]

        @pl.when(tcv % 2 == 0)
        def _():
            pltpu.async_copy(src, tiles.at[0], sem0)

        @pl.when(tcv % 2 == 1)
        def _():
            pltpu.async_copy(src, tiles.at[1], sem1)

    def wait_tile(tcv):
        @pl.when(tcv % 2 == 0)
        def _():
            pltpu.make_async_copy(
                wt_hbm.at[:, pl.ds(0, 128)], tiles.at[0], sem0
            ).wait()

        @pl.when(tcv % 2 == 1)
        def _():
            pltpu.make_async_copy(
                wt_hbm.at[:, pl.ds(0, 128)], tiles.at[1], sem1
            ).wait()

    def emit(q, col_vec_fn):
        @pl.when(q >= 8)
        def _():
            pltpu.make_async_copy(
                tmp.at[0], stage_hbm.at[0, pl.ds(0, DIM)], osem
            ).wait()

        slot = q % 8
        for k in range(4):
            tmp[slot, pl.ds(k * 16, 16)] = col_vec_fn(k)
        pltpu.async_copy(
            tmp.at[slot], stage_hbm.at[pval(q), pl.ds(0, DIM)], osem
        )

    s_first = sval(0)
    s_last = sval(L_PER_W - 1)
    tc_first = jnp.minimum(s_first >> 7, TC_EDGE - 1)
    tc_last = jnp.minimum(s_last >> 7, TC_EDGE - 1)
    have_main = s_first < EDGE0

    @pl.when(have_main)
    def _():
        fire(tc_first)

        def cond(c):
            _, tcv = c
            return tcv <= tc_last

        def body(c):
            p, tcv = c

            @pl.when(tcv + 1 <= tc_last)
            def _():
                fire(tcv + 1)

            wait_tile(tcv)
            par = tcv % 2

            def icond(q):
                v = sval(jnp.minimum(q, L_PER_W - 1))
                return (q < L_PER_W) & (v < EDGE0) & ((v >> 7) == tcv)

            def ibody(q):
                col = sval(q) & 127
                emit(
                    q,
                    lambda k: plsc.load_gather(
                        tiles,
                        [zero16 + par, lanes + k * 16, zero16 + col],
                    ),
                )
                return q + 1

            p = lax.while_loop(icond, ibody, p)
            return (p, tcv + 1)

        lax.while_loop(cond, body, (jnp.int32(0), tc_first))

    def find_edge_start(q, acc):
        v = sval(q)
        return jnp.where((v >= EDGE0) & (acc == L_PER_W), q, acc)

    p_edge = lax.fori_loop(0, L_PER_W, find_edge_start, jnp.int32(L_PER_W))

    def econd(q):
        return q < L_PER_W

    def ebody(q):
        col = sval(q) - EDGE0
        emit(
            q,
            lambda k: plsc.load_gather(ev, [lanes + k * 16, zero16 + col]),
        )
        return q + 1

    lax.while_loop(econd, ebody, p_edge)

    def dbody(_, c):
        pltpu.make_async_copy(
            tmp.at[0], stage_hbm.at[0, pl.ds(0, DIM)], osem
        ).wait()
        return c

    lax.fori_loop(0, 8, dbody, 0)


def _cos_body(stage_hbm, out_hbm, buf0, buf1, sums_v, out_v, sem0, sem1):
    wid = lax.axis_index("s") * 2 + lax.axis_index("c")
    base = wid * L_PER_W

    bufs = (buf0, buf1)
    sems = (sem0, sem1)

    def fetch(j):
        return pltpu.async_copy(
            stage_hbm.at[pl.ds(base + j * CHUNK2, CHUNK2)],
            bufs[j % 2],
            sems[j % 2],
        )

    zero16 = jnp.full((16,), 0, jnp.int32)
    lanes = lax.iota(jnp.int32, 16)
    last = lanes == 15

    def compute_chunk(j, buf):
        def e_body(i, _):
            p_acc = jnp.zeros((16,), jnp.float32)
            q_acc = jnp.zeros((16,), jnp.float32)
            r_acc = jnp.zeros((16,), jnp.float32)
            for k in range(4):
                a = buf[2 * i, pl.ds(k * 16, 16)]
                b = buf[2 * i + 1, pl.ds(k * 16, 16)]
                p_acc = p_acc + a * b
                q_acc = q_acc + a * a
                r_acc = r_acc + b * b
            ei = zero16 + (j * (CHUNK2 // 2) + i)
            plsc.store_scatter(sums_v, [ei], plsc.cumsum(p_acc), mask=last)
            plsc.store_scatter(
                sums_v, [ei + B_PER_W], plsc.cumsum(q_acc), mask=last)
            plsc.store_scatter(
                sums_v, [ei + 2 * B_PER_W], plsc.cumsum(r_acc), mask=last)
            return 0

        lax.fori_loop(0, CHUNK2 // 2, e_body, 0, unroll=2)

    copies = [fetch(0)]
    for j in range(NCHUNK2):
        if j + 1 < NCHUNK2:
            copies.append(fetch(j + 1))
        copies[j].wait()
        compute_chunk(j, bufs[j % 2])

    def blk_body(blk, _):
        sl = pl.ds(blk * 16, 16)
        s01 = sums_v[sl]
        s00 = sums_v[pl.ds(B_PER_W + blk * 16, 16)]
        s11 = sums_v[pl.ds(2 * B_PER_W + blk * 16, 16)]
        out_v[sl] = s01 * _rsqrt_newton(s00) * _rsqrt_newton(s11)
        return 0

    lax.fori_loop(0, NBLK, blk_body, 0)

    pltpu.sync_copy(out_v, out_hbm.at[pl.ds(wid * B_PER_W, B_PER_W)])


def kernel(x, W):
    xf = x.astype(jnp.int32).reshape(-1)
    pos = lax.iota(jnp.int32, 2 * BATCH)
    sv, pv = lax.sort((xf, pos), num_keys=1)
    sv3 = sv.reshape(NW, L_PER_W)
    pv3 = pv.reshape(NW, L_PER_W)
    wt = W.T
    wedge = jnp.pad(wt[:, EDGE0:], ((0, 0), (0, 128 - (NUMS - EDGE0))))

    mesh = plsc.VectorSubcoreMesh(core_axis_name="c", subcore_axis_name="s")
    params = pltpu.CompilerParams(
        needs_layout_passes=False, use_tc_tiling_on_sc=True
    )

    stage = pl.kernel(
        _gather_body,
        mesh=mesh,
        compiler_params=params,
        out_type=jax.ShapeDtypeStruct((2 * BATCH, 128), jnp.float32),
        scratch_types=[
            pltpu.VMEM((L_PER_W,), jnp.int32),
            pltpu.VMEM((L_PER_W,), jnp.int32),
            pltpu.VMEM((DIM, 128), jnp.float32),
            pltpu.VMEM((2, DIM, 128), jnp.float32),
            pltpu.VMEM((8, DIM), jnp.float32),
            pltpu.SemaphoreType.DMA,
            pltpu.SemaphoreType.DMA,
            pltpu.SemaphoreType.DMA,
        ],
    )(sv3, pv3, wt, wedge)

    out = pl.kernel(
        _cos_body,
        mesh=mesh,
        compiler_params=params,
        out_type=jax.ShapeDtypeStruct((BATCH,), jnp.float32),
        scratch_types=[
            pltpu.VMEM((CHUNK2, 128), jnp.float32),
            pltpu.VMEM((CHUNK2, 128), jnp.float32),
            pltpu.VMEM((3 * B_PER_W,), jnp.float32),
            pltpu.VMEM((B_PER_W,), jnp.float32),
            pltpu.SemaphoreType.DMA,
            pltpu.SemaphoreType.DMA,
        ],
    )(stage)
    return out[:, None]


# speedup vs baseline: 3.4911x; 1.4167x over previous
import jax
import jax.numpy as jnp
from jax import lax
from jax.experimental import pallas as pl
from jax.experimental.pallas import tpu as pltpu
from jax.experimental.pallas import tpu_sc as plsc

BATCH = 16384
DIM = 64
NUMS = 1000000
NW = 32
B_PER_W = BATCH // NW
L_PER_W = 2 * B_PER_W
TC_EDGE = NUMS // 128
EDGE0 = TC_EDGE * 128
CHUNK2 = 256
NCHUNK2 = L_PER_W // CHUNK2
NBLK = B_PER_W // 16


def _rsqrt_newton(s):
    i = plsc.bitcast(s, jnp.int32)
    y = plsc.bitcast(jnp.int32(0x5F3759DF) - (i >> 1), jnp.float32)
    half = s * 0.5
    for _ in range(3):
        y = y * (1.5 - half * y * y)
    return jnp.minimum(y, 1e12)


def _gather_body(sv_hbm, pv_hbm, wt_hbm, wedge_hbm, stage_hbm,
                 sv, pv, ev, tiles, tmp, sem0, sem1, sem2, sem3, osem):
    wid = lax.axis_index("s") * 2 + lax.axis_index("c")

    pltpu.sync_copy(sv_hbm.at[wid], sv)
    pltpu.sync_copy(pv_hbm.at[wid], pv)
    pltpu.sync_copy(wedge_hbm, ev)

    zero16 = jnp.full((16,), 0, jnp.int32)
    lanes = lax.iota(jnp.int32, 16)

    def sval(q):
        return plsc.load_gather(sv, [zero16 + q])[0]

    def pval(q):
        return plsc.load_gather(pv, [zero16 + q])[0]

    sems = (sem0, sem1, sem2, sem3)

    def fire(tcv):
        src = wt_hbm.at[:, pl.ds(pl.multiple_of(tcv * 128, 128), 128)]
        for b in range(4):
            @pl.when(tcv % 4 == b)
            def _(b=b):
                pltpu.async_copy(src, tiles.at[b], sems[b])

    def wait_tile(tcv):
        for b in range(4):
            @pl.when(tcv % 4 == b)
            def _(b=b):
                pltpu.make_async_copy(
                    wt_hbm.at[:, pl.ds(0, 128)], tiles.at[b], sems[b]
                ).wait()

    def emit(q, col_vec_fn):
        @pl.when(q >= 8)
        def _():
            pltpu.make_async_copy(
                tmp.at[0], stage_hbm.at[0, pl.ds(0, DIM)], osem
            ).wait()

        slot = q % 8
        for k in range(4):
            tmp[slot, pl.ds(k * 16, 16)] = col_vec_fn(k)
        pltpu.async_copy(
            tmp.at[slot], stage_hbm.at[pval(q), pl.ds(0, DIM)], osem
        )

    s_first = sval(0)
    s_last = sval(L_PER_W - 1)
    tc_first = jnp.minimum(s_first >> 7, TC_EDGE - 1)
    tc_last = jnp.minimum(s_last >> 7, TC_EDGE - 1)
    have_main = s_first < EDGE0

    @pl.when(have_main)
    def _():
        for d in range(4):
            @pl.when(tc_first + d <= tc_last)
            def _(d=d):
                fire(tc_first + d)

        def cond(c):
            _, tcv = c
            return tcv <= tc_last

        def body(c):
            p, tcv = c
            wait_tile(tcv)

            @pl.when(tcv + 4 <= tc_last)
            def _():
                fire(tcv + 4)

            par = tcv % 4

            def icond(q):
                v = sval(jnp.minimum(q, L_PER_W - 1))
                return (q < L_PER_W) & (v < EDGE0) & ((v >> 7) == tcv)

            def ibody(q):
                col = sval(q) & 127
                emit(
                    q,
                    lambda k: plsc.load_gather(
                        tiles,
                        [zero16 + par, lanes + k * 16, zero16 + col],
                    ),
                )
                return q + 1

            p = lax.while_loop(icond, ibody, p)
            return (p, tcv + 1)

        lax.while_loop(cond, body, (jnp.int32(0), tc_first))

    def find_edge_start(q, acc):
        v = sval(q)
        return jnp.where((v >= EDGE0) & (acc == L_PER_W), q, acc)

    p_edge = lax.fori_loop(0, L_PER_W, find_edge_start, jnp.int32(L_PER_W))

    def econd(q):
        return q < L_PER_W

    def ebody(q):
        col = sval(q) - EDGE0
        emit(
            q,
            lambda k: plsc.load_gather(ev, [lanes + k * 16, zero16 + col]),
        )
        return q + 1

    lax.while_loop(econd, ebody, p_edge)

    def dbody(_, c):
        pltpu.make_async_copy(
            tmp.at[0], stage_hbm.at[0, pl.ds(0, DIM)], osem
        ).wait()
        return c

    lax.fori_loop(0, 8, dbody, 0)


def _cos_body(stage_hbm, out_hbm, buf0, buf1, sums_v, out_v, sem0, sem1):
    wid = lax.axis_index("s") * 2 + lax.axis_index("c")
    base = wid * L_PER_W

    bufs = (buf0, buf1)
    sems = (sem0, sem1)

    def fetch(j):
        return pltpu.async_copy(
            stage_hbm.at[pl.ds(base + j * CHUNK2, CHUNK2)],
            bufs[j % 2],
            sems[j % 2],
        )

    zero16 = jnp.full((16,), 0, jnp.int32)
    lanes = lax.iota(jnp.int32, 16)
    last = lanes == 15

    def compute_chunk(j, buf):
        def e_body(i, _):
            p_acc = jnp.zeros((16,), jnp.float32)
            q_acc = jnp.zeros((16,), jnp.float32)
            r_acc = jnp.zeros((16,), jnp.float32)
            for k in range(4):
                a = buf[2 * i, pl.ds(k * 16, 16)]
                b = buf[2 * i + 1, pl.ds(k * 16, 16)]
                p_acc = p_acc + a * b
                q_acc = q_acc + a * a
                r_acc = r_acc + b * b
            ei = zero16 + (j * (CHUNK2 // 2) + i)
            plsc.store_scatter(sums_v, [ei], plsc.cumsum(p_acc), mask=last)
            plsc.store_scatter(
                sums_v, [ei + B_PER_W], plsc.cumsum(q_acc), mask=last)
            plsc.store_scatter(
                sums_v, [ei + 2 * B_PER_W], plsc.cumsum(r_acc), mask=last)
            return 0

        lax.fori_loop(0, CHUNK2 // 2, e_body, 0, unroll=2)

    copies = [fetch(0)]
    for j in range(NCHUNK2):
        if j + 1 < NCHUNK2:
            copies.append(fetch(j + 1))
        copies[j].wait()
        compute_chunk(j, bufs[j % 2])

    def blk_body(blk, _):
        sl = pl.ds(blk * 16, 16)
        s01 = sums_v[sl]
        s00 = sums_v[pl.ds(B_PER_W + blk * 16, 16)]
        s11 = sums_v[pl.ds(2 * B_PER_W + blk * 16, 16)]
        out_v[sl] = s01 * _rsqrt_newton(s00) * _rsqrt_newton(s11)
        return 0

    lax.fori_loop(0, NBLK, blk_body, 0)

    pltpu.sync_copy(out_v, out_hbm.at[pl.ds(wid * B_PER_W, B_PER_W)])


def kernel(x, W):
    xf = x.astype(jnp.int32).reshape(-1)
    pos = lax.iota(jnp.int32, 2 * BATCH)
    sv, pv = lax.sort((xf, pos), num_keys=1)
    sv3 = sv.reshape(NW, L_PER_W)
    pv3 = pv.reshape(NW, L_PER_W)
    wt = W.T
    wedge = jnp.pad(wt[:, EDGE0:], ((0, 0), (0, 128 - (NUMS - EDGE0))))

    mesh = plsc.VectorSubcoreMesh(core_axis_name="c", subcore_axis_name="s")
    params = pltpu.CompilerParams(
        needs_layout_passes=False, use_tc_tiling_on_sc=True
    )

    stage = pl.kernel(
        _gather_body,
        mesh=mesh,
        compiler_params=params,
        out_type=jax.ShapeDtypeStruct((2 * BATCH, 128), jnp.float32),
        scratch_types=[
            pltpu.VMEM((L_PER_W,), jnp.int32),
            pltpu.VMEM((L_PER_W,), jnp.int32),
            pltpu.VMEM((DIM, 128), jnp.float32),
            pltpu.VMEM((4, DIM, 128), jnp.float32),
            pltpu.VMEM((8, DIM), jnp.float32),
            pltpu.SemaphoreType.DMA,
            pltpu.SemaphoreType.DMA,
            pltpu.SemaphoreType.DMA,
            pltpu.SemaphoreType.DMA,
            pltpu.SemaphoreType.DMA,
        ],
    )(sv3, pv3, wt, wedge)

    out = pl.kernel(
        _cos_body,
        mesh=mesh,
        compiler_params=params,
        out_type=jax.ShapeDtypeStruct((BATCH,), jnp.float32),
        scratch_types=[
            pltpu.VMEM((CHUNK2, 128), jnp.float32),
            pltpu.VMEM((CHUNK2, 128), jnp.float32),
            pltpu.VMEM((3 * B_PER_W,), jnp.float32),
            pltpu.VMEM((B_PER_W,), jnp.float32),
            pltpu.SemaphoreType.DMA,
            pltpu.SemaphoreType.DMA,
        ],
    )(stage)
    return out[:, None]
